# baseline (speedup 1.0000x reference)
import jax
import jax.numpy as jnp
import numpy as np
from jax import lax
from jax.experimental import pallas as pl
from jax.experimental.pallas import tpu as pltpu
from jax.experimental.pallas import tpu_sc as plsc

VOCAB = 100000
SENT = 200
DIM = 64
BATCH = 1024

NC = 2
NS = 16
NW = NC * NS
LANES = 16

SENT_PER_W = BATCH // NW
CHUNK_S = 2
CHUNK = CHUNK_S * SENT
N_CHUNKS = SENT_PER_W // CHUNK_S
SUB = 80
N_SUB = CHUNK // SUB
D_SLICES = DIM // LANES
NBUF = 3
SCALE = 8.0


def _positional_encoding(length, depth):
    depth = depth / 2
    positions = np.arange(length)[:, np.newaxis]
    depths = np.arange(depth)[np.newaxis, :] / depth
    angle_rates = 1 / 10000 ** depths
    angle_rads = positions * angle_rates
    return np.concatenate([np.sin(angle_rads), np.cos(angle_rads)], axis=-1).astype(np.float32)

_POS_NP = _positional_encoding(SENT, DIM)


def _sc_body(table_hbm, x_hbm, pos_hbm, emb_hbm, mask_hbm,
             pos_v, idx_v, rows_v, mask_v, gsems, ssems):
    wid = lax.axis_index("s") * NC + lax.axis_index("c")
    s_base = wid * SENT_PER_W

    pltpu.sync_copy(pos_hbm, pos_v)

    def gather_descs(b):
        return [pltpu.make_async_copy(
                    table_hbm.at[idx_v[b].at[pl.ds(k * SUB, SUB)]],
                    rows_v[b].at[pl.ds(k * SUB, SUB)],
                    gsems[b]) for k in range(N_SUB)]

    def store_descs(c, b):
        s0 = s_base + c * CHUNK_S
        descs = []
        for j in range(CHUNK_S):
            descs.append(pltpu.make_async_copy(
                rows_v[b].at[pl.ds(j * SENT, SENT)], emb_hbm.at[s0 + j], ssems[b]))
            descs.append(pltpu.make_async_copy(
                mask_v[b].at[pl.ds(j * SENT, SENT)], mask_hbm.at[s0 + j], ssems[b]))
        return descs

    def fire_fetch(c, b):
        s0 = s_base + c * CHUNK_S
        for j in range(CHUNK_S):
            pltpu.sync_copy(x_hbm.at[s0 + j], idx_v[b].at[pl.ds(j * SENT, SENT)])
        for d in gather_descs(b):
            d.start()

    def compute(b):
        def mask_body(m, _):
            sl = pl.ds(m * LANES, LANES)
            mask_v[b][sl] = jnp.where(idx_v[b][sl] != 0, 1, 0).astype(jnp.int32)
            return _
        lax.fori_loop(0, CHUNK // LANES, mask_body, 0)

        def row_body(r, _):
            for d in range(D_SLICES):
                sl = pl.ds(d * LANES, LANES)
                pv = pos_v[r, sl]
                rows_v[b][r, sl] = rows_v[b][r, sl] * SCALE + pv
                rows_v[b][r + SENT, sl] = rows_v[b][r + SENT, sl] * SCALE + pv
            return _
        lax.fori_loop(0, SENT, row_body, 0)

    for c in range(N_CHUNKS):
        b = c % NBUF
        if c >= NBUF:
            for d in store_descs(c - NBUF, b):
                d.wait()
        fire_fetch(c, b)
        if c >= 1:
            bp = (c - 1) % NBUF
            for d in gather_descs(bp):
                d.wait()
            compute(bp)
            for d in store_descs(c - 1, bp):
                d.start()

    bl = (N_CHUNKS - 1) % NBUF
    for d in gather_descs(bl):
        d.wait()
    compute(bl)
    for d in store_descs(N_CHUNKS - 1, bl):
        d.start()
    for c in range(N_CHUNKS - NBUF, N_CHUNKS):
        for d in store_descs(c, c % NBUF):
            d.wait()


@jax.jit
def _decode_embedding(x, table, pos):
    mesh = plsc.VectorSubcoreMesh(
        core_axis_name="c", subcore_axis_name="s",
        num_cores=NC, num_subcores=NS)
    run = pl.kernel(
        _sc_body,
        out_type=(
            jax.ShapeDtypeStruct((BATCH, SENT, DIM), jnp.float32),
            jax.ShapeDtypeStruct((BATCH, SENT), jnp.int32),
        ),
        mesh=mesh,
        scratch_types=[
            pltpu.VMEM((SENT, DIM), jnp.float32),
            [pltpu.VMEM((CHUNK,), jnp.int32) for _ in range(NBUF)],
            [pltpu.VMEM((CHUNK, DIM), jnp.float32) for _ in range(NBUF)],
            [pltpu.VMEM((CHUNK,), jnp.int32) for _ in range(NBUF)],
            [pltpu.SemaphoreType.DMA for _ in range(NBUF)],
            [pltpu.SemaphoreType.DMA for _ in range(NBUF)],
        ],
        compiler_params=pltpu.CompilerParams(use_tc_tiling_on_sc=False),
    )
    return run(table, x, pos)


def kernel(x, embedding_table):
    pos = jnp.asarray(_POS_NP)
    return _decode_embedding(x, embedding_table, pos)
